# baseline (speedup 1.0000x reference)
import jax
import jax.numpy as jnp
from jax import lax
from jax.experimental import pallas as pl
from jax.experimental.pallas import tpu as pltpu
from jax.experimental.pallas import tpu_sc as plsc

N = 10000
C = 128
E = 320000
NOSC = 4
EPS_SPHERE = 1e-6
EPS_GN = 1e-5

NC = 2
NS = 16
NW = NC * NS
CH = 128
NCHUNK = 80
EPW = NCHUNK * CH
E_PAD = (NW + 1) * EPW
RING = 2
NP = 10240

_F32 = jnp.float32


def _agg_body(hs_hbm, src3_hbm, dst3_hbm, zer_hbm, out_hbm,
              sidx0, sidx1, dst_buf, rows0, rows1,
              acc_sh, gsem0, gsem1, ssem0, ssem1, isem0, isem1):
    cid = lax.axis_index("c")
    sid = lax.axis_index("s")
    wid = cid * NS + sid
    rows = [rows0, rows1]
    sidx = [sidx0, sidx1]
    gsem = [gsem0, gsem1]
    ssem = [ssem0, ssem1]
    isem = [isem0, isem1]

    @pl.when(sid == 0)
    def _():
        pltpu.sync_copy(zer_hbm, acc_sh)

    pltpu.sync_copy(dst3_hbm.at[wid], dst_buf)
    for r in range(RING):
        pltpu.sync_copy(src3_hbm.at[wid, r], sidx[r])
    plsc.subcore_barrier()

    for r in range(RING):
        pltpu.async_copy(hs_hbm.at[sidx[r]], rows[r], gsem[r])

    def body(k, carry):
        for r in range(RING):
            g = k * RING + r
            pltpu.make_async_copy(hs_hbm.at[sidx[r]], rows[r],
                                  gsem[r]).wait()
            pltpu.async_copy(rows[r], acc_sh.at[dst_buf.at[g]], ssem[r],
                             add=True)
            nxt = g + RING
            w2 = wid + nxt // NCHUNK
            g2 = lax.rem(nxt, NCHUNK)
            pltpu.async_copy(src3_hbm.at[w2, g2], sidx[r], isem[r])
        for r in range(RING):
            g = k * RING + r
            pltpu.make_async_copy(rows[r], acc_sh.at[dst_buf.at[g]],
                                  ssem[r]).wait()
            pltpu.make_async_copy(src3_hbm.at[wid, 0], sidx[r],
                                  isem[r]).wait()
            pltpu.async_copy(hs_hbm.at[sidx[r]], rows[r], gsem[r])
        return carry

    lax.fori_loop(0, NCHUNK // RING, body, 0)
    for r in range(RING):
        pltpu.make_async_copy(hs_hbm.at[sidx[r]], rows[r], gsem[r]).wait()
    plsc.subcore_barrier()

    @pl.when(sid == 0)
    def _():
        pltpu.sync_copy(acc_sh, out_hbm.at[cid])


_agg_call = pl.kernel(
    _agg_body,
    out_type=jax.ShapeDtypeStruct((NC, NP, C), _F32),
    mesh=plsc.VectorSubcoreMesh(core_axis_name="c", subcore_axis_name="s"),
    scratch_types=[
        pltpu.VMEM((CH,), jnp.int32),
        pltpu.VMEM((CH,), jnp.int32),
        pltpu.VMEM((NCHUNK, CH), jnp.int32),
        pltpu.VMEM((CH, C), _F32),
        pltpu.VMEM((CH, C), _F32),
        pltpu.VMEM_SHARED((NP, C), _F32),
        pltpu.SemaphoreType.DMA,
        pltpu.SemaphoreType.DMA,
        pltpu.SemaphoreType.DMA,
        pltpu.SemaphoreType.DMA,
        pltpu.SemaphoreType.DMA,
        pltpu.SemaphoreType.DMA,
    ],
)


def _gmat():
    ii = lax.broadcasted_iota(jnp.int32, (C, C), 0) // NOSC
    jj = lax.broadcasted_iota(jnp.int32, (C, C), 1) // NOSC
    return (ii == jj).astype(_F32)


def _gdot(p, G):
    return jnp.dot(p, G, precision=lax.Precision.HIGHEST,
                   preferred_element_type=_F32)


def _sphere(v, G):
    n2 = jnp.clip(_gdot(v * v, G), EPS_SPHERE, None)
    return v * lax.rsqrt(n2)


BNP = 2048


def _stats_body(y_ref, colsum_ref, colsq_ref):
    i = pl.program_id(0)
    y = y_ref[...]
    s1 = jnp.sum(y, axis=0, keepdims=True)
    s2 = jnp.sum(y * y, axis=0, keepdims=True)

    @pl.when(i == 0)
    def _():
        colsum_ref[...] = s1
        colsq_ref[...] = s2

    @pl.when(i != 0)
    def _():
        colsum_ref[...] += s1
        colsq_ref[...] += s2


_stats_call = pl.pallas_call(
    _stats_body,
    grid=(NP // BNP,),
    in_specs=[pl.BlockSpec((BNP, C), lambda i: (i, 0))],
    out_specs=[pl.BlockSpec((1, C), lambda i: (0, 0)),
               pl.BlockSpec((1, C), lambda i: (0, 0))],
    out_shape=[jax.ShapeDtypeStruct((1, C), _F32),
               jax.ShapeDtypeStruct((1, C), _F32)],
)


def _pre_body(y_ref, x_ref, degp_ref, colsum_ref, colsq_ref, gnw_ref, gnb_ref,
              w_ref, y2_ref, xs0_ref, hs0_ref, disc_ref):
    G = _gmat()
    cnt = _F32(NOSC * N)
    mean = _gdot(colsum_ref[...], G) / cnt
    var = _gdot(colsq_ref[...], G) / cnt - mean * mean
    yn = (y_ref[...] - mean) * lax.rsqrt(var + EPS_GN)
    yv = yn * gnw_ref[...] + gnb_ref[...]
    y2_ref[...] = _sphere(yv, G)

    xs0 = _sphere(x_ref[...], G)
    xs0_ref[...] = xs0

    deg = degp_ref[0][:, 0:1] + degp_ref[1][:, 0:1] + 1.0
    disc = jnp.broadcast_to(lax.rsqrt(deg), (BNP, C))
    disc_ref[...] = disc
    hs0_ref[...] = jnp.dot(xs0, w_ref[...], preferred_element_type=_F32) * disc


_prow_spec = pl.BlockSpec((BNP, C), lambda i: (i, 0))
_pre_call = pl.pallas_call(
    _pre_body,
    grid=(NP // BNP,),
    in_specs=[
        _prow_spec,
        _prow_spec,
        pl.BlockSpec((NC, BNP, C), lambda i: (0, i, 0)),
        pl.BlockSpec((1, C), lambda i: (0, 0)),
        pl.BlockSpec((1, C), lambda i: (0, 0)),
        pl.BlockSpec((1, C), lambda i: (0, 0)),
        pl.BlockSpec((1, C), lambda i: (0, 0)),
        pl.BlockSpec((C, C), lambda i: (0, 0)),
    ],
    out_specs=[_prow_spec, _prow_spec, _prow_spec, _prow_spec],
    out_shape=[
        jax.ShapeDtypeStruct((NP, C), _F32),
        jax.ShapeDtypeStruct((NP, C), _F32),
        jax.ShapeDtypeStruct((NP, C), _F32),
        jax.ShapeDtypeStruct((NP, C), _F32),
    ],
)


BN = 2048


def _step_body(xs_ref, aggp_ref, hs_ref, disc_ref, y2_ref, w_ref, b_ref,
               gam_ref, xsn_ref, hsn_ref):
    G = _gmat()
    xs = xs_ref[...]
    dis = disc_ref[...]
    c = dis * (aggp_ref[0] + aggp_ref[1] + hs_ref[...]) + b_ref[...] + y2_ref[...]
    sim = _gdot(xs * c, G)
    dxdt = c - sim * xs
    xn = xs + gam_ref[...] * dxdt
    xsn = _sphere(xn, G)
    xsn_ref[...] = xsn
    hsn_ref[...] = jnp.dot(xsn, w_ref[...], preferred_element_type=_F32) * dis


_row_spec = pl.BlockSpec((BN, C), lambda i: (i, 0))
_step_call = pl.pallas_call(
    _step_body,
    grid=(NP // BN,),
    in_specs=[
        _row_spec,
        pl.BlockSpec((NC, BN, C), lambda i: (0, i, 0)),
        _row_spec,
        _row_spec,
        _row_spec,
        pl.BlockSpec((C, C), lambda i: (0, 0)),
        pl.BlockSpec((1, C), lambda i: (0, 0)),
        pl.BlockSpec((1, 1), lambda i: (0, 0)),
    ],
    out_specs=[_row_spec, _row_spec],
    out_shape=[
        jax.ShapeDtypeStruct((NP, C), _F32),
        jax.ShapeDtypeStruct((NP, C), _F32),
    ],
)


def kernel(x, y, sc, Q, gamma, W_gcn, b_gcn, gn_weight, gn_bias):
    pad = jnp.zeros((NP - N, C), _F32)
    x2 = jnp.concatenate([x.reshape(N, C), pad])
    y2in = jnp.concatenate([y.reshape(N, C), pad])
    epad = E_PAD - E
    src3 = jnp.concatenate([sc[0], jnp.zeros((epad,), sc.dtype)]).reshape(
        NW + 1, NCHUNK, CH)
    dst3 = jnp.concatenate([sc[1], jnp.full((epad,), NP - 1, sc.dtype)]).reshape(
        NW + 1, NCHUNK, CH)
    ones_tab = jnp.ones((NP, C), _F32)
    zer_agg = jnp.zeros((NP, C), _F32)
    gnw = gn_weight.reshape(1, C)
    gnb = gn_bias.reshape(1, C)
    bb = b_gcn.reshape(1, C)
    gam = jnp.asarray(gamma, _F32).reshape(1, 1)

    degp = _agg_call(ones_tab, src3, dst3, zer_agg)
    colsum, colsq = _stats_call(y2in)
    y2n, xs0, hs0, disc = _pre_call(y2in, x2, degp, colsum, colsq, gnw, gnb, W_gcn)

    def body(_, carry):
        xs, hs = carry
        aggp = _agg_call(hs, src3, dst3, zer_agg)
        xsn, hsn = _step_call(xs, aggp, hs, disc, y2n, W_gcn, bb, gam)
        return (xsn, hsn)

    xs, _ = lax.fori_loop(0, Q, body, (xs0, hs0))
    return xs[:N].reshape(1, N, C)


# speedup vs baseline: 1.7030x; 1.7030x over previous
import jax
import jax.numpy as jnp
from jax import lax
from jax.experimental import pallas as pl
from jax.experimental.pallas import tpu as pltpu
from jax.experimental.pallas import tpu_sc as plsc

N = 10000
C = 128
E = 320000
NOSC = 4
EPS_SPHERE = 1e-6
EPS_GN = 1e-5

NC = 2
NS = 16
NW = NC * NS
CH = 128
NCHUNK = 79
EPW = NCHUNK * CH
TAIL = 2
E_PAD = NW * EPW + TAIL * CH
IDX_SHIFT = 14
NP = 10240

_F32 = jnp.float32


def _agg_body(hs_hbm, pk_hbm, zer_hbm, out_hbm,
              pk_buf, sidx0, sidx1, didx0, didx1, rows0, rows1,
              acc_sh, gsem0, gsem1, ssem0, ssem1):
    cid = lax.axis_index("c")
    sid = lax.axis_index("s")
    wid = cid * NS + sid
    sidx = [sidx0, sidx1]
    didx = [didx0, didx1]
    rows = [rows0, rows1]
    gsem = [gsem0, gsem1]
    ssem = [ssem0, ssem1]

    @pl.when(sid == 0)
    def _():
        pltpu.sync_copy(zer_hbm, acc_sh)

    pltpu.sync_copy(pk_hbm.at[pl.ds(wid * EPW, EPW + TAIL * CH)], pk_buf)
    plsc.subcore_barrier()

    def unpack(g, r):
        for j in range(CH // 16):
            v = pk_buf[pl.ds(g * CH + 16 * j, 16)]
            sidx[r][pl.ds(16 * j, 16)] = lax.shift_right_logical(v, IDX_SHIFT)
            didx[r][pl.ds(16 * j, 16)] = v & ((1 << IDX_SHIFT) - 1)

    def gather(g, r):
        pltpu.async_copy(hs_hbm.at[sidx[r]], rows[r], gsem[r])

    def gather_wait(r):
        pltpu.make_async_copy(hs_hbm.at[sidx[r]], rows[r], gsem[r]).wait()

    def scatter(r):
        pltpu.async_copy(rows[r], acc_sh.at[didx[r]], ssem[r], add=True)

    def scatter_wait(r):
        pltpu.make_async_copy(rows[r], acc_sh.at[didx[r]], ssem[r]).wait()

    unpack(0, 0)
    gather(0, 0)
    unpack(1, 1)
    gather(1, 1)
    gather_wait(0)
    scatter(0)

    def body(k, carry):
        g = 2 * k + 1
        gather_wait(1)
        scatter(1)
        scatter_wait(0)
        unpack(g + 1, 0)
        gather(g + 1, 0)
        gather_wait(0)
        scatter(0)
        scatter_wait(1)
        unpack(g + 2, 1)
        gather(g + 2, 1)
        return carry

    lax.fori_loop(0, (NCHUNK - 1) // 2, body, 0)
    pltpu.make_async_copy(hs_hbm.at[sidx[1]], rows[1], gsem[1]).wait()
    pltpu.make_async_copy(rows[0], acc_sh.at[didx[0]], ssem[0]).wait()
    plsc.subcore_barrier()

    @pl.when(sid == 0)
    def _():
        pltpu.sync_copy(acc_sh, out_hbm.at[cid])


_agg_call = pl.kernel(
    _agg_body,
    out_type=jax.ShapeDtypeStruct((NC, NP, C), _F32),
    mesh=plsc.VectorSubcoreMesh(core_axis_name="c", subcore_axis_name="s"),
    scratch_types=[
        pltpu.VMEM((EPW + TAIL * CH,), jnp.int32),
        pltpu.VMEM((CH,), jnp.int32),
        pltpu.VMEM((CH,), jnp.int32),
        pltpu.VMEM((CH,), jnp.int32),
        pltpu.VMEM((CH,), jnp.int32),
        pltpu.VMEM((CH, C), _F32),
        pltpu.VMEM((CH, C), _F32),
        pltpu.VMEM_SHARED((NP, C), _F32),
        pltpu.SemaphoreType.DMA,
        pltpu.SemaphoreType.DMA,
        pltpu.SemaphoreType.DMA,
        pltpu.SemaphoreType.DMA,
    ],
)


def _gmat():
    ii = lax.broadcasted_iota(jnp.int32, (C, C), 0) // NOSC
    jj = lax.broadcasted_iota(jnp.int32, (C, C), 1) // NOSC
    return (ii == jj).astype(_F32)


def _gdot(p, G):
    return jnp.dot(p, G, precision=lax.Precision.HIGHEST,
                   preferred_element_type=_F32)


def _sphere(v, G):
    n2 = jnp.clip(_gdot(v * v, G), EPS_SPHERE, None)
    return v * lax.rsqrt(n2)


BNP = 2048


def _stats_body(y_ref, colsum_ref, colsq_ref):
    i = pl.program_id(0)
    y = y_ref[...]
    s1 = jnp.sum(y, axis=0, keepdims=True)
    s2 = jnp.sum(y * y, axis=0, keepdims=True)

    @pl.when(i == 0)
    def _():
        colsum_ref[...] = s1
        colsq_ref[...] = s2

    @pl.when(i != 0)
    def _():
        colsum_ref[...] += s1
        colsq_ref[...] += s2


_stats_call = pl.pallas_call(
    _stats_body,
    grid=(NP // BNP,),
    in_specs=[pl.BlockSpec((BNP, C), lambda i: (i, 0))],
    out_specs=[pl.BlockSpec((1, C), lambda i: (0, 0)),
               pl.BlockSpec((1, C), lambda i: (0, 0))],
    out_shape=[jax.ShapeDtypeStruct((1, C), _F32),
               jax.ShapeDtypeStruct((1, C), _F32)],
)


def _pre_body(y_ref, x_ref, degp_ref, colsum_ref, colsq_ref, gnw_ref, gnb_ref,
              w_ref, y2_ref, xs0_ref, hs0_ref, disc_ref):
    G = _gmat()
    cnt = _F32(NOSC * N)
    mean = _gdot(colsum_ref[...], G) / cnt
    var = _gdot(colsq_ref[...], G) / cnt - mean * mean
    yn = (y_ref[...] - mean) * lax.rsqrt(var + EPS_GN)
    yv = yn * gnw_ref[...] + gnb_ref[...]
    y2_ref[...] = _sphere(yv, G)

    xs0 = _sphere(x_ref[...], G)
    xs0_ref[...] = xs0

    deg = degp_ref[0][:, 0:1] + degp_ref[1][:, 0:1] + 1.0
    disc = jnp.broadcast_to(lax.rsqrt(deg), (BNP, C))
    disc_ref[...] = disc
    hs0_ref[...] = jnp.dot(xs0, w_ref[...], preferred_element_type=_F32) * disc


_prow_spec = pl.BlockSpec((BNP, C), lambda i: (i, 0))
_pre_call = pl.pallas_call(
    _pre_body,
    grid=(NP // BNP,),
    in_specs=[
        _prow_spec,
        _prow_spec,
        pl.BlockSpec((NC, BNP, C), lambda i: (0, i, 0)),
        pl.BlockSpec((1, C), lambda i: (0, 0)),
        pl.BlockSpec((1, C), lambda i: (0, 0)),
        pl.BlockSpec((1, C), lambda i: (0, 0)),
        pl.BlockSpec((1, C), lambda i: (0, 0)),
        pl.BlockSpec((C, C), lambda i: (0, 0)),
    ],
    out_specs=[_prow_spec, _prow_spec, _prow_spec, _prow_spec],
    out_shape=[
        jax.ShapeDtypeStruct((NP, C), _F32),
        jax.ShapeDtypeStruct((NP, C), _F32),
        jax.ShapeDtypeStruct((NP, C), _F32),
        jax.ShapeDtypeStruct((NP, C), _F32),
    ],
)


BN = 2048


def _step_body(xs_ref, aggp_ref, hs_ref, disc_ref, y2_ref, w_ref, b_ref,
               gam_ref, xsn_ref, hsn_ref):
    G = _gmat()
    xs = xs_ref[...]
    dis = disc_ref[...]
    c = dis * (aggp_ref[0] + aggp_ref[1] + hs_ref[...]) + b_ref[...] + y2_ref[...]
    sim = _gdot(xs * c, G)
    dxdt = c - sim * xs
    xn = xs + gam_ref[...] * dxdt
    xsn = _sphere(xn, G)
    xsn_ref[...] = xsn
    hsn_ref[...] = jnp.dot(xsn, w_ref[...], preferred_element_type=_F32) * dis


_row_spec = pl.BlockSpec((BN, C), lambda i: (i, 0))
_step_call = pl.pallas_call(
    _step_body,
    grid=(NP // BN,),
    in_specs=[
        _row_spec,
        pl.BlockSpec((NC, BN, C), lambda i: (0, i, 0)),
        _row_spec,
        _row_spec,
        _row_spec,
        pl.BlockSpec((C, C), lambda i: (0, 0)),
        pl.BlockSpec((1, C), lambda i: (0, 0)),
        pl.BlockSpec((1, 1), lambda i: (0, 0)),
    ],
    out_specs=[_row_spec, _row_spec],
    out_shape=[
        jax.ShapeDtypeStruct((NP, C), _F32),
        jax.ShapeDtypeStruct((NP, C), _F32),
    ],
)


def kernel(x, y, sc, Q, gamma, W_gcn, b_gcn, gn_weight, gn_bias):
    pad = jnp.zeros((NP - N, C), _F32)
    x2 = jnp.concatenate([x.reshape(N, C), pad])
    y2in = jnp.concatenate([y.reshape(N, C), pad])
    epad = E_PAD - E
    packed = jnp.concatenate([
        (sc[0] << IDX_SHIFT) | sc[1],
        jnp.full((epad,), NP - 1, sc.dtype),
    ])
    ones_tab = jnp.ones((NP, C), _F32)
    zer_agg = jnp.zeros((NP, C), _F32)
    gnw = gn_weight.reshape(1, C)
    gnb = gn_bias.reshape(1, C)
    bb = b_gcn.reshape(1, C)
    gam = jnp.asarray(gamma, _F32).reshape(1, 1)

    degp = _agg_call(ones_tab, packed, zer_agg)
    colsum, colsq = _stats_call(y2in)
    y2n, xs0, hs0, disc = _pre_call(y2in, x2, degp, colsum, colsq, gnw, gnb, W_gcn)

    def body(_, carry):
        xs, hs = carry
        aggp = _agg_call(hs, packed, zer_agg)
        xsn, hsn = _step_call(xs, aggp, hs, disc, y2n, W_gcn, bb, gam)
        return (xsn, hsn)

    xs, _ = lax.fori_loop(0, Q, body, (xs0, hs0))
    return xs[:N].reshape(1, N, C)


# speedup vs baseline: 1.8873x; 1.1083x over previous
import jax
import jax.numpy as jnp
from jax import lax
from jax.experimental import pallas as pl
from jax.experimental.pallas import tpu as pltpu
from jax.experimental.pallas import tpu_sc as plsc

N = 10000
C = 128
E = 320000
NOSC = 4
EPS_SPHERE = 1e-6
EPS_GN = 1e-5

NC = 2
NS = 16
NW = NC * NS
CH = 128
NCHUNK0 = 107
NCHUNK1 = 51
EPW0 = NCHUNK0 * CH
EPW1 = NCHUNK1 * CH
TAIL = 2
E_PAD = NS * (EPW0 + EPW1) + EPW0 + TAIL * CH
IDX_SHIFT = 14
NP = 10240

_F32 = jnp.float32


def _agg_body(hs_hbm, pk_hbm, zer_hbm, out_hbm,
              pk_buf, sidx0, sidx1, didx0, didx1, rows0, rows1,
              acc_sh, gsem0, gsem1, ssem0, ssem1):
    cid = lax.axis_index("c")
    sid = lax.axis_index("s")
    nchunk = jnp.where(cid == 0, NCHUNK0, NCHUNK1)
    base = cid * (NS * EPW0) + sid * jnp.where(cid == 0, EPW0, EPW1)
    sidx = [sidx0, sidx1]
    didx = [didx0, didx1]
    rows = [rows0, rows1]
    gsem = [gsem0, gsem1]
    ssem = [ssem0, ssem1]

    @pl.when(sid == 0)
    def _():
        pltpu.sync_copy(zer_hbm, acc_sh)

    pltpu.sync_copy(pk_hbm.at[pl.ds(base, EPW0 + TAIL * CH)], pk_buf)
    plsc.subcore_barrier()

    def unpack(g, r):
        for j in range(CH // 16):
            v = pk_buf[pl.ds(g * CH + 16 * j, 16)]
            sidx[r][pl.ds(16 * j, 16)] = lax.shift_right_logical(v, IDX_SHIFT)
            didx[r][pl.ds(16 * j, 16)] = v & ((1 << IDX_SHIFT) - 1)

    def gather(g, r):
        pltpu.async_copy(hs_hbm.at[sidx[r]], rows[r], gsem[r])

    def gather_wait(r):
        pltpu.make_async_copy(hs_hbm.at[sidx[r]], rows[r], gsem[r]).wait()

    def scatter(r):
        pltpu.async_copy(rows[r], acc_sh.at[didx[r]], ssem[r], add=True)

    def scatter_wait(r):
        pltpu.make_async_copy(rows[r], acc_sh.at[didx[r]], ssem[r]).wait()

    unpack(0, 0)
    gather(0, 0)
    unpack(1, 1)
    gather(1, 1)
    gather_wait(0)
    scatter(0)

    def body(k, carry):
        g = 2 * k + 1
        gather_wait(1)
        scatter(1)
        scatter_wait(0)
        unpack(g + 1, 0)
        gather(g + 1, 0)
        gather_wait(0)
        scatter(0)
        scatter_wait(1)
        unpack(g + 2, 1)
        gather(g + 2, 1)
        return carry

    lax.fori_loop(0, (nchunk - 1) // 2, body, 0)
    pltpu.make_async_copy(hs_hbm.at[sidx[1]], rows[1], gsem[1]).wait()
    pltpu.make_async_copy(rows[0], acc_sh.at[didx[0]], ssem[0]).wait()
    plsc.subcore_barrier()

    @pl.when(sid == 0)
    def _():
        pltpu.sync_copy(acc_sh, out_hbm.at[cid])


_agg_call = pl.kernel(
    _agg_body,
    out_type=jax.ShapeDtypeStruct((NC, NP, C), _F32),
    mesh=plsc.VectorSubcoreMesh(core_axis_name="c", subcore_axis_name="s"),
    scratch_types=[
        pltpu.VMEM((EPW0 + TAIL * CH,), jnp.int32),
        pltpu.VMEM((CH,), jnp.int32),
        pltpu.VMEM((CH,), jnp.int32),
        pltpu.VMEM((CH,), jnp.int32),
        pltpu.VMEM((CH,), jnp.int32),
        pltpu.VMEM((CH, C), _F32),
        pltpu.VMEM((CH, C), _F32),
        pltpu.VMEM_SHARED((NP, C), _F32),
        pltpu.SemaphoreType.DMA,
        pltpu.SemaphoreType.DMA,
        pltpu.SemaphoreType.DMA,
        pltpu.SemaphoreType.DMA,
    ],
)


def _gmat():
    ii = lax.broadcasted_iota(jnp.int32, (C, C), 0) // NOSC
    jj = lax.broadcasted_iota(jnp.int32, (C, C), 1) // NOSC
    return (ii == jj).astype(_F32)


def _gdot(p, G):
    return jnp.dot(p, G, precision=lax.Precision.HIGHEST,
                   preferred_element_type=_F32)


def _sphere(v, G):
    n2 = jnp.clip(_gdot(v * v, G), EPS_SPHERE, None)
    return v * lax.rsqrt(n2)


BNP = 2048


def _stats_body(y_ref, colsum_ref, colsq_ref):
    i = pl.program_id(0)
    y = y_ref[...]
    s1 = jnp.sum(y, axis=0, keepdims=True)
    s2 = jnp.sum(y * y, axis=0, keepdims=True)

    @pl.when(i == 0)
    def _():
        colsum_ref[...] = s1
        colsq_ref[...] = s2

    @pl.when(i != 0)
    def _():
        colsum_ref[...] += s1
        colsq_ref[...] += s2


_stats_call = pl.pallas_call(
    _stats_body,
    grid=(NP // BNP,),
    in_specs=[pl.BlockSpec((BNP, C), lambda i: (i, 0))],
    out_specs=[pl.BlockSpec((1, C), lambda i: (0, 0)),
               pl.BlockSpec((1, C), lambda i: (0, 0))],
    out_shape=[jax.ShapeDtypeStruct((1, C), _F32),
               jax.ShapeDtypeStruct((1, C), _F32)],
)


def _pre_body(y_ref, x_ref, degp_ref, colsum_ref, colsq_ref, gnw_ref, gnb_ref,
              w_ref, y2_ref, xs0_ref, hs0_ref, disc_ref):
    G = _gmat()
    cnt = _F32(NOSC * N)
    mean = _gdot(colsum_ref[...], G) / cnt
    var = _gdot(colsq_ref[...], G) / cnt - mean * mean
    yn = (y_ref[...] - mean) * lax.rsqrt(var + EPS_GN)
    yv = yn * gnw_ref[...] + gnb_ref[...]
    y2_ref[...] = _sphere(yv, G)

    xs0 = _sphere(x_ref[...], G)
    xs0_ref[...] = xs0

    deg = degp_ref[0][:, 0:1] + degp_ref[1][:, 0:1] + 1.0
    disc = jnp.broadcast_to(lax.rsqrt(deg), (BNP, C))
    disc_ref[...] = disc
    hs0_ref[...] = jnp.dot(xs0, w_ref[...], preferred_element_type=_F32) * disc


_prow_spec = pl.BlockSpec((BNP, C), lambda i: (i, 0))
_pre_call = pl.pallas_call(
    _pre_body,
    grid=(NP // BNP,),
    in_specs=[
        _prow_spec,
        _prow_spec,
        pl.BlockSpec((NC, BNP, C), lambda i: (0, i, 0)),
        pl.BlockSpec((1, C), lambda i: (0, 0)),
        pl.BlockSpec((1, C), lambda i: (0, 0)),
        pl.BlockSpec((1, C), lambda i: (0, 0)),
        pl.BlockSpec((1, C), lambda i: (0, 0)),
        pl.BlockSpec((C, C), lambda i: (0, 0)),
    ],
    out_specs=[_prow_spec, _prow_spec, _prow_spec, _prow_spec],
    out_shape=[
        jax.ShapeDtypeStruct((NP, C), _F32),
        jax.ShapeDtypeStruct((NP, C), _F32),
        jax.ShapeDtypeStruct((NP, C), _F32),
        jax.ShapeDtypeStruct((NP, C), _F32),
    ],
)


BN = 2048


def _step_body(xs_ref, aggp_ref, hs_ref, disc_ref, y2_ref, w_ref, b_ref,
               gam_ref, xsn_ref, hsn_ref):
    G = _gmat()
    xs = xs_ref[...]
    dis = disc_ref[...]
    c = dis * (aggp_ref[0] + aggp_ref[1] + hs_ref[...]) + b_ref[...] + y2_ref[...]
    sim = _gdot(xs * c, G)
    dxdt = c - sim * xs
    xn = xs + gam_ref[...] * dxdt
    xsn = _sphere(xn, G)
    xsn_ref[...] = xsn
    hsn_ref[...] = jnp.dot(xsn, w_ref[...], preferred_element_type=_F32) * dis


_row_spec = pl.BlockSpec((BN, C), lambda i: (i, 0))
_step_call = pl.pallas_call(
    _step_body,
    grid=(NP // BN,),
    in_specs=[
        _row_spec,
        pl.BlockSpec((NC, BN, C), lambda i: (0, i, 0)),
        _row_spec,
        _row_spec,
        _row_spec,
        pl.BlockSpec((C, C), lambda i: (0, 0)),
        pl.BlockSpec((1, C), lambda i: (0, 0)),
        pl.BlockSpec((1, 1), lambda i: (0, 0)),
    ],
    out_specs=[_row_spec, _row_spec],
    out_shape=[
        jax.ShapeDtypeStruct((NP, C), _F32),
        jax.ShapeDtypeStruct((NP, C), _F32),
    ],
)


def kernel(x, y, sc, Q, gamma, W_gcn, b_gcn, gn_weight, gn_bias):
    pad = jnp.zeros((NP - N, C), _F32)
    x2 = jnp.concatenate([x.reshape(N, C), pad])
    y2in = jnp.concatenate([y.reshape(N, C), pad])
    epad = E_PAD - E
    packed = jnp.concatenate([
        (sc[0] << IDX_SHIFT) | sc[1],
        jnp.full((epad,), NP - 1, sc.dtype),
    ])
    ones_tab = jnp.ones((NP, C), _F32)
    zer_agg = jnp.zeros((NP, C), _F32)
    gnw = gn_weight.reshape(1, C)
    gnb = gn_bias.reshape(1, C)
    bb = b_gcn.reshape(1, C)
    gam = jnp.asarray(gamma, _F32).reshape(1, 1)

    degp = _agg_call(ones_tab, packed, zer_agg)
    colsum, colsq = _stats_call(y2in)
    y2n, xs0, hs0, disc = _pre_call(y2in, x2, degp, colsum, colsq, gnw, gnb, W_gcn)

    def body(_, carry):
        xs, hs = carry
        aggp = _agg_call(hs, packed, zer_agg)
        xsn, hsn = _step_call(xs, aggp, hs, disc, y2n, W_gcn, bb, gam)
        return (xsn, hsn)

    xs, _ = lax.fori_loop(0, Q, body, (xs0, hs0))
    return xs[:N].reshape(1, N, C)


# speedup vs baseline: 2.4508x; 1.2985x over previous
import jax
import jax.numpy as jnp
from jax import lax
from jax.experimental import pallas as pl
from jax.experimental.pallas import tpu as pltpu
from jax.experimental.pallas import tpu_sc as plsc

N = 10000
C = 128
E = 320000
NOSC = 4
EPS_SPHERE = 1e-6
EPS_GN = 1e-5

NC = 2
NS = 16
NW = NC * NS
CH = 128
NCHUNK0 = 119
NCHUNK1 = 39
EPW0 = NCHUNK0 * CH
EPW1 = NCHUNK1 * CH
TAIL = 2
E_PAD = NS * (EPW0 + EPW1) + EPW0 + TAIL * CH
IDX_SHIFT = 14
NP = 10240

_F32 = jnp.float32


def _agg_body(hs_hbm, pk_hbm, zer_hbm, out_hbm,
              pk_buf, sidx0, sidx1, didx0, didx1, rows0, rows1,
              acc_sh, gsem0, gsem1, ssem0, ssem1):
    cid = lax.axis_index("c")
    sid = lax.axis_index("s")
    nchunk = jnp.where(cid == 0, NCHUNK0, NCHUNK1)
    base = cid * (NS * EPW0) + sid * jnp.where(cid == 0, EPW0, EPW1)
    sidx = [sidx0, sidx1]
    didx = [didx0, didx1]
    rows = [rows0, rows1]
    gsem = [gsem0, gsem1]
    ssem = [ssem0, ssem1]

    @pl.when(sid == 0)
    def _():
        pltpu.sync_copy(zer_hbm, acc_sh)

    pltpu.sync_copy(pk_hbm.at[pl.ds(base, EPW0 + TAIL * CH)], pk_buf)
    plsc.subcore_barrier()

    def unpack(g, r):
        for j in range(CH // 16):
            v = pk_buf[pl.ds(g * CH + 16 * j, 16)]
            sidx[r][pl.ds(16 * j, 16)] = lax.shift_right_logical(v, IDX_SHIFT)
            didx[r][pl.ds(16 * j, 16)] = v & ((1 << IDX_SHIFT) - 1)

    def gather(g, r):
        pltpu.async_copy(hs_hbm.at[sidx[r]], rows[r], gsem[r])

    def gather_wait(r):
        pltpu.make_async_copy(hs_hbm.at[sidx[r]], rows[r], gsem[r]).wait()

    def scatter(r):
        pltpu.async_copy(rows[r], acc_sh.at[didx[r]], ssem[r], add=True)

    def scatter_wait(r):
        pltpu.make_async_copy(rows[r], acc_sh.at[didx[r]], ssem[r]).wait()

    unpack(0, 0)
    gather(0, 0)
    unpack(1, 1)
    gather(1, 1)
    gather_wait(0)
    scatter(0)

    def body(k, carry):
        g = 2 * k + 1
        gather_wait(1)
        scatter(1)
        scatter_wait(0)
        unpack(g + 1, 0)
        gather(g + 1, 0)
        gather_wait(0)
        scatter(0)
        scatter_wait(1)
        unpack(g + 2, 1)
        gather(g + 2, 1)
        return carry

    lax.fori_loop(0, (nchunk - 1) // 2, body, 0)
    pltpu.make_async_copy(hs_hbm.at[sidx[1]], rows[1], gsem[1]).wait()
    pltpu.make_async_copy(rows[0], acc_sh.at[didx[0]], ssem[0]).wait()
    plsc.subcore_barrier()

    @pl.when(sid == 0)
    def _():
        pltpu.sync_copy(acc_sh, out_hbm.at[cid])


_agg_call = pl.kernel(
    _agg_body,
    out_type=jax.ShapeDtypeStruct((NC, NP, C), _F32),
    mesh=plsc.VectorSubcoreMesh(core_axis_name="c", subcore_axis_name="s"),
    scratch_types=[
        pltpu.VMEM((EPW0 + TAIL * CH,), jnp.int32),
        pltpu.VMEM((CH,), jnp.int32),
        pltpu.VMEM((CH,), jnp.int32),
        pltpu.VMEM((CH,), jnp.int32),
        pltpu.VMEM((CH,), jnp.int32),
        pltpu.VMEM((CH, C), _F32),
        pltpu.VMEM((CH, C), _F32),
        pltpu.VMEM_SHARED((NP, C), _F32),
        pltpu.SemaphoreType.DMA,
        pltpu.SemaphoreType.DMA,
        pltpu.SemaphoreType.DMA,
        pltpu.SemaphoreType.DMA,
    ],
)


def _deg_body(ones_hbm, pk_hbm, zer_hbm, out_hbm,
              pk_buf, didx0, didx1, ones_v,
              acc_sh, ssem0, ssem1):
    cid = lax.axis_index("c")
    sid = lax.axis_index("s")
    nchunk = jnp.where(cid == 0, NCHUNK0, NCHUNK1)
    base = cid * (NS * EPW0) + sid * jnp.where(cid == 0, EPW0, EPW1)
    didx = [didx0, didx1]
    ssem = [ssem0, ssem1]

    @pl.when(sid == 0)
    def _():
        pltpu.sync_copy(zer_hbm, acc_sh)

    pltpu.sync_copy(pk_hbm.at[pl.ds(base, EPW0 + TAIL * CH)], pk_buf)
    pltpu.sync_copy(ones_hbm, ones_v)
    plsc.subcore_barrier()

    def unpack(g, r):
        for j in range(CH // 16):
            v = pk_buf[pl.ds(g * CH + 16 * j, 16)]
            didx[r][pl.ds(16 * j, 16)] = v & ((1 << IDX_SHIFT) - 1)

    def scatter(r):
        pltpu.async_copy(ones_v, acc_sh.at[didx[r]], ssem[r], add=True)

    def scatter_wait(r):
        pltpu.make_async_copy(ones_v, acc_sh.at[didx[r]], ssem[r]).wait()

    unpack(0, 0)
    scatter(0)
    unpack(1, 1)
    scatter(1)

    def body(k, carry):
        g = 2 * k
        scatter_wait(0)
        unpack(g + 2, 0)
        scatter(0)
        scatter_wait(1)
        unpack(g + 3, 1)
        scatter(1)
        return carry

    lax.fori_loop(0, (nchunk - 3) // 2, body, 0)
    scatter_wait(0)
    unpack(nchunk - 1, 0)
    scatter(0)
    scatter_wait(0)
    scatter_wait(1)
    plsc.subcore_barrier()

    @pl.when(sid == 0)
    def _():
        pltpu.sync_copy(acc_sh, out_hbm.at[cid])


_deg_call = pl.kernel(
    _deg_body,
    out_type=jax.ShapeDtypeStruct((NC, NP, C), _F32),
    mesh=plsc.VectorSubcoreMesh(core_axis_name="c", subcore_axis_name="s"),
    scratch_types=[
        pltpu.VMEM((EPW0 + TAIL * CH,), jnp.int32),
        pltpu.VMEM((CH,), jnp.int32),
        pltpu.VMEM((CH,), jnp.int32),
        pltpu.VMEM((CH, C), _F32),
        pltpu.VMEM_SHARED((NP, C), _F32),
        pltpu.SemaphoreType.DMA,
        pltpu.SemaphoreType.DMA,
    ],
)


def _gmat():
    ii = lax.broadcasted_iota(jnp.int32, (C, C), 0) // NOSC
    jj = lax.broadcasted_iota(jnp.int32, (C, C), 1) // NOSC
    return (ii == jj).astype(_F32)


def _gdot(p, G):
    return jnp.dot(p, G, precision=lax.Precision.HIGHEST,
                   preferred_element_type=_F32)


def _sphere(v, G):
    n2 = jnp.clip(_gdot(v * v, G), EPS_SPHERE, None)
    return v * lax.rsqrt(n2)


BNP = 2048


def _stats_body(y_ref, colsum_ref, colsq_ref):
    i = pl.program_id(0)
    y = y_ref[...]
    s1 = jnp.sum(y, axis=0, keepdims=True)
    s2 = jnp.sum(y * y, axis=0, keepdims=True)

    @pl.when(i == 0)
    def _():
        colsum_ref[...] = s1
        colsq_ref[...] = s2

    @pl.when(i != 0)
    def _():
        colsum_ref[...] += s1
        colsq_ref[...] += s2


_stats_call = pl.pallas_call(
    _stats_body,
    grid=(NP // BNP,),
    in_specs=[pl.BlockSpec((BNP, C), lambda i: (i, 0))],
    out_specs=[pl.BlockSpec((1, C), lambda i: (0, 0)),
               pl.BlockSpec((1, C), lambda i: (0, 0))],
    out_shape=[jax.ShapeDtypeStruct((1, C), _F32),
               jax.ShapeDtypeStruct((1, C), _F32)],
)


def _pre_body(y_ref, x_ref, degp_ref, colsum_ref, colsq_ref, gnw_ref, gnb_ref,
              w_ref, y2_ref, xs0_ref, hs0_ref, disc_ref):
    G = _gmat()
    cnt = _F32(NOSC * N)
    mean = _gdot(colsum_ref[...], G) / cnt
    var = _gdot(colsq_ref[...], G) / cnt - mean * mean
    yn = (y_ref[...] - mean) * lax.rsqrt(var + EPS_GN)
    yv = yn * gnw_ref[...] + gnb_ref[...]
    y2_ref[...] = _sphere(yv, G)

    xs0 = _sphere(x_ref[...], G)
    xs0_ref[...] = xs0

    deg = degp_ref[0][:, 0:1] + degp_ref[1][:, 0:1] + 1.0
    disc = jnp.broadcast_to(lax.rsqrt(deg), (BNP, C))
    disc_ref[...] = disc
    hs0_ref[...] = jnp.dot(xs0, w_ref[...], preferred_element_type=_F32) * disc


_prow_spec = pl.BlockSpec((BNP, C), lambda i: (i, 0))
_pre_call = pl.pallas_call(
    _pre_body,
    grid=(NP // BNP,),
    in_specs=[
        _prow_spec,
        _prow_spec,
        pl.BlockSpec((NC, BNP, C), lambda i: (0, i, 0)),
        pl.BlockSpec((1, C), lambda i: (0, 0)),
        pl.BlockSpec((1, C), lambda i: (0, 0)),
        pl.BlockSpec((1, C), lambda i: (0, 0)),
        pl.BlockSpec((1, C), lambda i: (0, 0)),
        pl.BlockSpec((C, C), lambda i: (0, 0)),
    ],
    out_specs=[_prow_spec, _prow_spec, _prow_spec, _prow_spec],
    out_shape=[
        jax.ShapeDtypeStruct((NP, C), _F32),
        jax.ShapeDtypeStruct((NP, C), _F32),
        jax.ShapeDtypeStruct((NP, C), _F32),
        jax.ShapeDtypeStruct((NP, C), _F32),
    ],
)


BN = 2048


def _step_body(xs_ref, aggp_ref, hs_ref, disc_ref, y2_ref, w_ref, b_ref,
               gam_ref, xsn_ref, hsn_ref):
    G = _gmat()
    xs = xs_ref[...]
    dis = disc_ref[...]
    c = dis * (aggp_ref[0] + aggp_ref[1] + hs_ref[...]) + b_ref[...] + y2_ref[...]
    sim = _gdot(xs * c, G)
    dxdt = c - sim * xs
    xn = xs + gam_ref[...] * dxdt
    xsn = _sphere(xn, G)
    xsn_ref[...] = xsn
    hsn_ref[...] = jnp.dot(xsn, w_ref[...], preferred_element_type=_F32) * dis


_row_spec = pl.BlockSpec((BN, C), lambda i: (i, 0))
_step_call = pl.pallas_call(
    _step_body,
    grid=(NP // BN,),
    in_specs=[
        _row_spec,
        pl.BlockSpec((NC, BN, C), lambda i: (0, i, 0)),
        _row_spec,
        _row_spec,
        _row_spec,
        pl.BlockSpec((C, C), lambda i: (0, 0)),
        pl.BlockSpec((1, C), lambda i: (0, 0)),
        pl.BlockSpec((1, 1), lambda i: (0, 0)),
    ],
    out_specs=[_row_spec, _row_spec],
    out_shape=[
        jax.ShapeDtypeStruct((NP, C), _F32),
        jax.ShapeDtypeStruct((NP, C), _F32),
    ],
)


def kernel(x, y, sc, Q, gamma, W_gcn, b_gcn, gn_weight, gn_bias):
    pad = jnp.zeros((NP - N, C), _F32)
    x2 = jnp.concatenate([x.reshape(N, C), pad])
    y2in = jnp.concatenate([y.reshape(N, C), pad])
    epad = E_PAD - E
    packed = jnp.concatenate([
        (sc[0] << IDX_SHIFT) | sc[1],
        jnp.full((epad,), NP - 1, sc.dtype),
    ])
    ones_row = jnp.ones((CH, C), _F32)
    zer_agg = jnp.zeros((NP, C), _F32)
    gnw = gn_weight.reshape(1, C)
    gnb = gn_bias.reshape(1, C)
    bb = b_gcn.reshape(1, C)
    gam = jnp.asarray(gamma, _F32).reshape(1, 1)

    degp = _deg_call(ones_row, packed, zer_agg)
    colsum, colsq = _stats_call(y2in)
    y2n, xs0, hs0, disc = _pre_call(y2in, x2, degp, colsum, colsq, gnw, gnb, W_gcn)

    def body(_, carry):
        xs, hs = carry
        aggp = _agg_call(hs, packed, zer_agg)
        xsn, hsn = _step_call(xs, aggp, hs, disc, y2n, W_gcn, bb, gam)
        return (xsn, hsn)

    xs, _ = lax.fori_loop(0, Q, body, (xs0, hs0))
    return xs[:N].reshape(1, N, C)


# speedup vs baseline: 2.4695x; 1.0076x over previous
import jax
import jax.numpy as jnp
from jax import lax
from jax.experimental import pallas as pl
from jax.experimental.pallas import tpu as pltpu
from jax.experimental.pallas import tpu_sc as plsc

N = 10000
C = 128
E = 320000
NOSC = 4
EPS_SPHERE = 1e-6
EPS_GN = 1e-5

NC = 2
NS = 16
NW = NC * NS
CH = 128
NCHUNK0 = 121
NCHUNK1 = 37
EPW0 = NCHUNK0 * CH
EPW1 = NCHUNK1 * CH
TAIL = 2
E_PAD = NS * (EPW0 + EPW1) + EPW0 + TAIL * CH
IDX_SHIFT = 14
NP = 10240

_F32 = jnp.float32


def _agg_body(hs_hbm, pk_hbm, zer_hbm, out_hbm,
              pk_buf, sidx0, sidx1, didx0, didx1, rows0, rows1,
              acc_sh, gsem0, gsem1, ssem0, ssem1):
    cid = lax.axis_index("c")
    sid = lax.axis_index("s")
    nchunk = jnp.where(cid == 0, NCHUNK0, NCHUNK1)
    base = cid * (NS * EPW0) + sid * jnp.where(cid == 0, EPW0, EPW1)
    sidx = [sidx0, sidx1]
    didx = [didx0, didx1]
    rows = [rows0, rows1]
    gsem = [gsem0, gsem1]
    ssem = [ssem0, ssem1]

    @pl.when(sid == 0)
    def _():
        pltpu.sync_copy(zer_hbm, acc_sh)

    pltpu.sync_copy(pk_hbm.at[pl.ds(base, EPW0 + TAIL * CH)], pk_buf)
    plsc.subcore_barrier()

    def unpack(g, r):
        for j in range(CH // 16):
            v = pk_buf[pl.ds(g * CH + 16 * j, 16)]
            sidx[r][pl.ds(16 * j, 16)] = lax.shift_right_logical(v, IDX_SHIFT)
            didx[r][pl.ds(16 * j, 16)] = v & ((1 << IDX_SHIFT) - 1)

    def gather(g, r):
        pltpu.async_copy(hs_hbm.at[sidx[r]], rows[r], gsem[r])

    def gather_wait(r):
        pltpu.make_async_copy(hs_hbm.at[sidx[r]], rows[r], gsem[r]).wait()

    def scatter(r):
        pltpu.async_copy(rows[r], acc_sh.at[didx[r]], ssem[r], add=True)

    def scatter_wait(r):
        pltpu.make_async_copy(rows[r], acc_sh.at[didx[r]], ssem[r]).wait()

    unpack(0, 0)
    gather(0, 0)
    unpack(1, 1)
    gather(1, 1)
    gather_wait(0)
    scatter(0)

    def body(k, carry):
        g = 2 * k + 1
        gather_wait(1)
        scatter(1)
        scatter_wait(0)
        unpack(g + 1, 0)
        gather(g + 1, 0)
        gather_wait(0)
        scatter(0)
        scatter_wait(1)
        unpack(g + 2, 1)
        gather(g + 2, 1)
        return carry

    lax.fori_loop(0, (nchunk - 1) // 2, body, 0)
    pltpu.make_async_copy(hs_hbm.at[sidx[1]], rows[1], gsem[1]).wait()
    pltpu.make_async_copy(rows[0], acc_sh.at[didx[0]], ssem[0]).wait()
    plsc.subcore_barrier()

    @pl.when(sid == 0)
    def _():
        pltpu.sync_copy(acc_sh, out_hbm.at[cid])


_agg_call = pl.kernel(
    _agg_body,
    out_type=jax.ShapeDtypeStruct((NC, NP, C), _F32),
    mesh=plsc.VectorSubcoreMesh(core_axis_name="c", subcore_axis_name="s"),
    scratch_types=[
        pltpu.VMEM((EPW0 + TAIL * CH,), jnp.int32),
        pltpu.VMEM((CH,), jnp.int32),
        pltpu.VMEM((CH,), jnp.int32),
        pltpu.VMEM((CH,), jnp.int32),
        pltpu.VMEM((CH,), jnp.int32),
        pltpu.VMEM((CH, C), _F32),
        pltpu.VMEM((CH, C), _F32),
        pltpu.VMEM_SHARED((NP, C), _F32),
        pltpu.SemaphoreType.DMA,
        pltpu.SemaphoreType.DMA,
        pltpu.SemaphoreType.DMA,
        pltpu.SemaphoreType.DMA,
    ],
)


def _deg_body(ones_hbm, pk_hbm, zer_hbm, out_hbm,
              pk_buf, didx0, didx1, ones_v,
              acc_sh, ssem0, ssem1):
    cid = lax.axis_index("c")
    sid = lax.axis_index("s")
    nchunk = jnp.where(cid == 0, NCHUNK0, NCHUNK1)
    base = cid * (NS * EPW0) + sid * jnp.where(cid == 0, EPW0, EPW1)
    didx = [didx0, didx1]
    ssem = [ssem0, ssem1]

    @pl.when(sid == 0)
    def _():
        pltpu.sync_copy(zer_hbm, acc_sh)

    pltpu.sync_copy(pk_hbm.at[pl.ds(base, EPW0 + TAIL * CH)], pk_buf)
    pltpu.sync_copy(ones_hbm, ones_v)
    plsc.subcore_barrier()

    def unpack(g, r):
        for j in range(CH // 16):
            v = pk_buf[pl.ds(g * CH + 16 * j, 16)]
            didx[r][pl.ds(16 * j, 16)] = v & ((1 << IDX_SHIFT) - 1)

    def scatter(r):
        pltpu.async_copy(ones_v, acc_sh.at[didx[r]], ssem[r], add=True)

    def scatter_wait(r):
        pltpu.make_async_copy(ones_v, acc_sh.at[didx[r]], ssem[r]).wait()

    unpack(0, 0)
    scatter(0)
    unpack(1, 1)
    scatter(1)

    def body(k, carry):
        g = 2 * k
        scatter_wait(0)
        unpack(g + 2, 0)
        scatter(0)
        scatter_wait(1)
        unpack(g + 3, 1)
        scatter(1)
        return carry

    lax.fori_loop(0, (nchunk - 3) // 2, body, 0)
    scatter_wait(0)
    unpack(nchunk - 1, 0)
    scatter(0)
    scatter_wait(0)
    scatter_wait(1)
    plsc.subcore_barrier()

    @pl.when(sid == 0)
    def _():
        pltpu.sync_copy(acc_sh, out_hbm.at[cid])


_deg_call = pl.kernel(
    _deg_body,
    out_type=jax.ShapeDtypeStruct((NC, NP, C), _F32),
    mesh=plsc.VectorSubcoreMesh(core_axis_name="c", subcore_axis_name="s"),
    scratch_types=[
        pltpu.VMEM((EPW0 + TAIL * CH,), jnp.int32),
        pltpu.VMEM((CH,), jnp.int32),
        pltpu.VMEM((CH,), jnp.int32),
        pltpu.VMEM((CH, C), _F32),
        pltpu.VMEM_SHARED((NP, C), _F32),
        pltpu.SemaphoreType.DMA,
        pltpu.SemaphoreType.DMA,
    ],
)


def _gmat():
    ii = lax.broadcasted_iota(jnp.int32, (C, C), 0) // NOSC
    jj = lax.broadcasted_iota(jnp.int32, (C, C), 1) // NOSC
    return (ii == jj).astype(_F32)


def _gdot(p, G):
    return jnp.dot(p, G, precision=lax.Precision.HIGHEST,
                   preferred_element_type=_F32)


def _sphere(v, G):
    n2 = jnp.clip(_gdot(v * v, G), EPS_SPHERE, None)
    return v * lax.rsqrt(n2)


BNP = 2048


def _stats_body(y_ref, colsum_ref, colsq_ref):
    i = pl.program_id(0)
    y = y_ref[...]
    s1 = jnp.sum(y, axis=0, keepdims=True)
    s2 = jnp.sum(y * y, axis=0, keepdims=True)

    @pl.when(i == 0)
    def _():
        colsum_ref[...] = s1
        colsq_ref[...] = s2

    @pl.when(i != 0)
    def _():
        colsum_ref[...] += s1
        colsq_ref[...] += s2


_stats_call = pl.pallas_call(
    _stats_body,
    grid=(NP // BNP,),
    in_specs=[pl.BlockSpec((BNP, C), lambda i: (i, 0))],
    out_specs=[pl.BlockSpec((1, C), lambda i: (0, 0)),
               pl.BlockSpec((1, C), lambda i: (0, 0))],
    out_shape=[jax.ShapeDtypeStruct((1, C), _F32),
               jax.ShapeDtypeStruct((1, C), _F32)],
)


def _pre_body(y_ref, x_ref, degp_ref, colsum_ref, colsq_ref, gnw_ref, gnb_ref,
              w_ref, y2_ref, xs0_ref, hs0_ref, disc_ref):
    G = _gmat()
    cnt = _F32(NOSC * N)
    mean = _gdot(colsum_ref[...], G) / cnt
    var = _gdot(colsq_ref[...], G) / cnt - mean * mean
    yn = (y_ref[...] - mean) * lax.rsqrt(var + EPS_GN)
    yv = yn * gnw_ref[...] + gnb_ref[...]
    y2_ref[...] = _sphere(yv, G)

    xs0 = _sphere(x_ref[...], G)
    xs0_ref[...] = xs0

    deg = degp_ref[0][:, 0:1] + degp_ref[1][:, 0:1] + 1.0
    disc = jnp.broadcast_to(lax.rsqrt(deg), (BNP, C))
    disc_ref[...] = disc
    hs0_ref[...] = jnp.dot(xs0, w_ref[...], preferred_element_type=_F32) * disc


_prow_spec = pl.BlockSpec((BNP, C), lambda i: (i, 0))
_pre_call = pl.pallas_call(
    _pre_body,
    grid=(NP // BNP,),
    in_specs=[
        _prow_spec,
        _prow_spec,
        pl.BlockSpec((NC, BNP, C), lambda i: (0, i, 0)),
        pl.BlockSpec((1, C), lambda i: (0, 0)),
        pl.BlockSpec((1, C), lambda i: (0, 0)),
        pl.BlockSpec((1, C), lambda i: (0, 0)),
        pl.BlockSpec((1, C), lambda i: (0, 0)),
        pl.BlockSpec((C, C), lambda i: (0, 0)),
    ],
    out_specs=[_prow_spec, _prow_spec, _prow_spec, _prow_spec],
    out_shape=[
        jax.ShapeDtypeStruct((NP, C), _F32),
        jax.ShapeDtypeStruct((NP, C), _F32),
        jax.ShapeDtypeStruct((NP, C), _F32),
        jax.ShapeDtypeStruct((NP, C), _F32),
    ],
)


BN = 2048


def _step_body(xs_ref, aggp_ref, hs_ref, disc_ref, y2_ref, w_ref, b_ref,
               gam_ref, xsn_ref, hsn_ref):
    G = _gmat()
    xs = xs_ref[...]
    dis = disc_ref[...]
    c = dis * (aggp_ref[0] + aggp_ref[1] + hs_ref[...]) + b_ref[...] + y2_ref[...]
    sim = _gdot(xs * c, G)
    dxdt = c - sim * xs
    xn = xs + gam_ref[...] * dxdt
    xsn = _sphere(xn, G)
    xsn_ref[...] = xsn
    hsn_ref[...] = jnp.dot(xsn, w_ref[...], preferred_element_type=_F32) * dis


_row_spec = pl.BlockSpec((BN, C), lambda i: (i, 0))
_step_call = pl.pallas_call(
    _step_body,
    grid=(NP // BN,),
    in_specs=[
        _row_spec,
        pl.BlockSpec((NC, BN, C), lambda i: (0, i, 0)),
        _row_spec,
        _row_spec,
        _row_spec,
        pl.BlockSpec((C, C), lambda i: (0, 0)),
        pl.BlockSpec((1, C), lambda i: (0, 0)),
        pl.BlockSpec((1, 1), lambda i: (0, 0)),
    ],
    out_specs=[_row_spec, _row_spec],
    out_shape=[
        jax.ShapeDtypeStruct((NP, C), _F32),
        jax.ShapeDtypeStruct((NP, C), _F32),
    ],
)


def kernel(x, y, sc, Q, gamma, W_gcn, b_gcn, gn_weight, gn_bias):
    pad = jnp.zeros((NP - N, C), _F32)
    x2 = jnp.concatenate([x.reshape(N, C), pad])
    y2in = jnp.concatenate([y.reshape(N, C), pad])
    epad = E_PAD - E
    packed = jnp.concatenate([
        (sc[0] << IDX_SHIFT) | sc[1],
        jnp.full((epad,), NP - 1, sc.dtype),
    ])
    ones_row = jnp.ones((CH, C), _F32)
    zer_agg = jnp.zeros((NP, C), _F32)
    gnw = gn_weight.reshape(1, C)
    gnb = gn_bias.reshape(1, C)
    bb = b_gcn.reshape(1, C)
    gam = jnp.asarray(gamma, _F32).reshape(1, 1)

    degp = _deg_call(ones_row, packed, zer_agg)
    colsum, colsq = _stats_call(y2in)
    y2n, xs0, hs0, disc = _pre_call(y2in, x2, degp, colsum, colsq, gnw, gnb, W_gcn)

    def body(_, carry):
        xs, hs = carry
        aggp = _agg_call(hs, packed, zer_agg)
        xsn, hsn = _step_call(xs, aggp, hs, disc, y2n, W_gcn, bb, gam)
        return (xsn, hsn)

    xs, _ = lax.fori_loop(0, Q, body, (xs0, hs0))
    return xs[:N].reshape(1, N, C)
